# baseline (speedup 1.0000x reference)
import functools

import jax
import jax.numpy as jnp
from jax.experimental import pallas as pl

_BIG = 1e30


def _body(ps_ref, pt_ref, nc_ref, x_ref, xs_ref, w_ref, b_ref, out_ref):
    ps = ps_ref[...]
    pt = pt_ref[0:3, :]
    ns = jnp.sum(ps * ps, axis=1, keepdims=True)
    dot = jax.lax.dot_general(ps, pt, (((1,), (0,)), ((), ())),
                              preferred_element_type=jnp.float32)
    d2 = jnp.maximum(ns + nc_ref[...] - 2.0 * dot, 0.0)
    m1 = jnp.min(d2, axis=1, keepdims=True)
    m2 = jnp.min(jnp.where(d2 <= m1, _BIG, d2), axis=1, keepdims=True)
    m3 = jnp.min(jnp.where(d2 <= m2, _BIG, d2), axis=1, keepdims=True)
    inv = 1.0 / jnp.maximum(d2, 1e-16)
    keep = d2 <= m3
    den = jnp.sum(jnp.where(keep, inv, 0.0), axis=1, keepdims=True)
    w = jnp.where(keep, inv, 0.0).astype(jnp.bfloat16)
    num = jax.lax.dot_general(w, x_ref[...], (((1,), (0,)), ((), ())),
                              preferred_element_type=jnp.float32)
    h = num / den
    w1 = w_ref[0:128, :]
    w2 = w_ref[128:192, :]
    acc = jax.lax.dot_general(h, w1, (((1,), (0,)), ((), ())),
                              preferred_element_type=jnp.float32,
                              precision=jax.lax.Precision.HIGHEST)
    acc += jax.lax.dot_general(xs_ref[...], w2, (((1,), (0,)), ((), ())),
                               preferred_element_type=jnp.float32,
                               precision=jax.lax.Precision.HIGHEST)
    out_ref[...] = jnp.maximum(acc + b_ref[...], 0.0)


@functools.partial(jax.jit, static_argnums=())
def kernel(x, pos, batch, x_skip, pos_skip, batch_skip, W, b):
    Nc, dx = x.shape
    Nf, dskip = x_skip.shape
    dout = W.shape[1]
    NCP = 5120
    F = 400

    pos_t = jnp.zeros((8, NCP), dtype=jnp.float32).at[:3, :Nc].set(pos.T)
    nc_row = jnp.full((1, NCP), 1e10, dtype=jnp.float32).at[0, :Nc].set(
        jnp.sum(pos * pos, axis=1))
    x_pad = jnp.zeros((NCP, dx), dtype=jnp.bfloat16).at[:Nc, :].set(
        x.astype(jnp.bfloat16))
    b2 = b.reshape(1, dout)

    grid = Nf // F
    out = pl.pallas_call(
        _body,
        grid=(grid,),
        in_specs=[
            pl.BlockSpec((F, 3), lambda i: (i, 0)),
            pl.BlockSpec((8, NCP), lambda i: (0, 0)),
            pl.BlockSpec((1, NCP), lambda i: (0, 0)),
            pl.BlockSpec((NCP, dx), lambda i: (0, 0)),
            pl.BlockSpec((F, dskip), lambda i: (i, 0)),
            pl.BlockSpec((dx + dskip, dout), lambda i: (0, 0)),
            pl.BlockSpec((1, dout), lambda i: (0, 0)),
        ],
        out_specs=pl.BlockSpec((F, dout), lambda i: (i, 0)),
        out_shape=jax.ShapeDtypeStruct((Nf, dout), jnp.float32),
    )(pos_skip, pos_t, nc_row, x_pad, x_skip, W, b2)
    return (out, pos_skip, batch_skip)


# speedup vs baseline: 1.0414x; 1.0414x over previous
import functools

import jax
import jax.numpy as jnp
from jax.experimental import pallas as pl

_BIG = 1e30


def _body(ps_ref, pt_ref, nc_ref, x_ref, xs_ref, w_ref, b_ref, out_ref):
    ps = ps_ref[...]
    pt = pt_ref[0:3, :]
    ns = jnp.sum(ps * ps, axis=1, keepdims=True)
    dot = jax.lax.dot_general(ps, pt, (((1,), (0,)), ((), ())),
                              preferred_element_type=jnp.float32)
    d2 = jnp.maximum(ns + nc_ref[...] - 2.0 * dot, 0.0)
    m1 = jnp.min(d2, axis=1, keepdims=True)
    m2 = jnp.min(jnp.where(d2 <= m1, _BIG, d2), axis=1, keepdims=True)
    m3 = jnp.min(jnp.where(d2 <= m2, _BIG, d2), axis=1, keepdims=True)
    w = jnp.where(d2 <= m3, 1.0 / jnp.maximum(d2, 1e-16), 0.0)
    den = jnp.sum(w, axis=1, keepdims=True)
    num = jax.lax.dot_general(w, x_ref[...], (((1,), (0,)), ((), ())),
                              preferred_element_type=jnp.float32)
    h = num / den
    w1 = w_ref[0:128, :]
    w2 = w_ref[128:192, :]
    acc = jax.lax.dot_general(h, w1, (((1,), (0,)), ((), ())),
                              preferred_element_type=jnp.float32,
                              precision=jax.lax.Precision.HIGHEST)
    acc += jax.lax.dot_general(xs_ref[...], w2, (((1,), (0,)), ((), ())),
                               preferred_element_type=jnp.float32,
                               precision=jax.lax.Precision.HIGHEST)
    out_ref[...] = jnp.maximum(acc + b_ref[...], 0.0)


@functools.partial(jax.jit, static_argnums=())
def kernel(x, pos, batch, x_skip, pos_skip, batch_skip, W, b):
    Nc, dx = x.shape
    Nf, dskip = x_skip.shape
    dout = W.shape[1]
    NCP = 5120
    F = 400

    pos_t = jnp.zeros((8, NCP), dtype=jnp.float32).at[:3, :Nc].set(pos.T)
    nc_row = jnp.full((1, NCP), 1e10, dtype=jnp.float32).at[0, :Nc].set(
        jnp.sum(pos * pos, axis=1))
    x_pad = jnp.zeros((NCP, dx), dtype=jnp.bfloat16).at[:Nc, :].set(
        x.astype(jnp.bfloat16))
    b2 = b.reshape(1, dout)

    grid = Nf // F
    out = pl.pallas_call(
        _body,
        grid=(grid,),
        in_specs=[
            pl.BlockSpec((F, 3), lambda i: (i, 0)),
            pl.BlockSpec((8, NCP), lambda i: (0, 0)),
            pl.BlockSpec((1, NCP), lambda i: (0, 0)),
            pl.BlockSpec((NCP, dx), lambda i: (0, 0)),
            pl.BlockSpec((F, dskip), lambda i: (i, 0)),
            pl.BlockSpec((dx + dskip, dout), lambda i: (0, 0)),
            pl.BlockSpec((1, dout), lambda i: (0, 0)),
        ],
        out_specs=pl.BlockSpec((F, dout), lambda i: (i, 0)),
        out_shape=jax.ShapeDtypeStruct((Nf, dout), jnp.float32),
    )(pos_skip, pos_t, nc_row, x_pad, x_skip, W, b2)
    return (out, pos_skip, batch_skip)


# speedup vs baseline: 1.0732x; 1.0305x over previous
import functools

import jax
import jax.numpy as jnp
from jax.experimental import pallas as pl

_BIG = 1e30


def _body(ps_ref, pt_ref, nc_ref, x_ref, xs_ref, w_ref, b_ref, out_ref):
    ps = ps_ref[...]
    pt = pt_ref[0:3, :]
    ns = jnp.sum(ps * ps, axis=1, keepdims=True)
    dot = jax.lax.dot_general(ps, pt, (((1,), (0,)), ((), ())),
                              preferred_element_type=jnp.float32)
    d2 = jnp.maximum(ns + nc_ref[...] - 2.0 * dot, 0.0)
    m1 = jnp.min(d2, axis=1, keepdims=True)
    m2 = jnp.min(jnp.where(d2 <= m1, _BIG, d2), axis=1, keepdims=True)
    m3 = jnp.min(jnp.where(d2 <= m2, _BIG, d2), axis=1, keepdims=True)
    w = jnp.where(d2 <= m3, 1.0 / jnp.maximum(d2, 1e-16), 0.0)
    den = jnp.sum(w, axis=1, keepdims=True)
    num = jax.lax.dot_general(w, x_ref[...], (((1,), (0,)), ((), ())),
                              preferred_element_type=jnp.float32)
    h = num / den
    w1 = w_ref[0:128, :]
    w2 = w_ref[128:192, :]
    acc = jax.lax.dot_general(h, w1, (((1,), (0,)), ((), ())),
                              preferred_element_type=jnp.float32,
                              precision=jax.lax.Precision.HIGHEST)
    acc += jax.lax.dot_general(xs_ref[...], w2, (((1,), (0,)), ((), ())),
                               preferred_element_type=jnp.float32,
                               precision=jax.lax.Precision.HIGHEST)
    out_ref[...] = jnp.maximum(acc + b_ref[...], 0.0)


@functools.partial(jax.jit, static_argnums=())
def kernel(x, pos, batch, x_skip, pos_skip, batch_skip, W, b):
    Nc, dx = x.shape
    Nf, dskip = x_skip.shape
    dout = W.shape[1]
    NCP = 5120
    F = 800

    pos_t = jnp.zeros((8, NCP), dtype=jnp.float32).at[:3, :Nc].set(pos.T)
    nc_row = jnp.full((1, NCP), 1e10, dtype=jnp.float32).at[0, :Nc].set(
        jnp.sum(pos * pos, axis=1))
    x_pad = jnp.zeros((NCP, dx), dtype=jnp.bfloat16).at[:Nc, :].set(
        x.astype(jnp.bfloat16))
    b2 = b.reshape(1, dout)

    grid = Nf // F
    out = pl.pallas_call(
        _body,
        grid=(grid,),
        in_specs=[
            pl.BlockSpec((F, 3), lambda i: (i, 0)),
            pl.BlockSpec((8, NCP), lambda i: (0, 0)),
            pl.BlockSpec((1, NCP), lambda i: (0, 0)),
            pl.BlockSpec((NCP, dx), lambda i: (0, 0)),
            pl.BlockSpec((F, dskip), lambda i: (i, 0)),
            pl.BlockSpec((dx + dskip, dout), lambda i: (0, 0)),
            pl.BlockSpec((1, dout), lambda i: (0, 0)),
        ],
        out_specs=pl.BlockSpec((F, dout), lambda i: (i, 0)),
        out_shape=jax.ShapeDtypeStruct((Nf, dout), jnp.float32),
    )(pos_skip, pos_t, nc_row, x_pad, x_skip, W, b2)
    return (out, pos_skip, batch_skip)
